# Initial kernel scaffold; baseline (speedup 1.0000x reference)
import jax
import jax.numpy as jnp
from jax.experimental import pallas as pl


def kernel(x, edge_index, train_mask, y, W1, a_s1, a_d1, b1, bn_gamma, bn_beta, bn_mean, bn_var, W2, a_s2, a_d2, b2, embedding_w, logit_p):
    raise NotImplementedError("write your pallas kernel here")


# speedup vs baseline: 14.2653x; 14.2653x over previous
import functools

import jax
import jax.numpy as jnp
import numpy as np
from jax import lax
from jax.experimental import pallas as pl
from jax.experimental.pallas import tpu as pltpu
from jax.experimental.pallas import tpu_sc as plsc

F32 = jnp.float32
I32 = jnp.int32
EPS = float(np.finfo(float).eps)

NC, NS = 2, 16
NW = NC * NS
CH = 128
RB = 128


def _rup(a, b):
    return (a + b - 1) // b * b


def _k1_body(n, x_ref, w1_ref, as1_ref, ad1_ref, lp_ref, ew_ref,
             h1e_ref, es_ref, ed_ref, emb_ref, s_ref):
    xb = x_ref[...]
    h1 = jnp.dot(xb, w1_ref[...], preferred_element_type=F32)
    es_ref[...] = jnp.dot(h1, as1_ref[...], preferred_element_type=F32)
    ed_ref[...] = jnp.dot(h1, ad1_ref[...], preferred_element_type=F32)
    h1e_ref[...] = h1
    p = jax.nn.sigmoid(lp_ref[...])
    kk = jnp.log(p + EPS) - jnp.log(1.0 - p + EPS)
    keep = 1.0 - jax.nn.sigmoid(kk / 0.1)
    ew = jax.nn.sigmoid(ew_ref[...])
    embb = xb * (keep * ew)
    emb_ref[...] = embb
    s_ref[...] = jnp.sum(embb, axis=1, keepdims=True)


def _run_k1(n, n_pad, in_c, hid_c, r1, x_p, W1, a_s1, a_d1, logit_p, embedding_w):
    grid = n_pad // RB
    return pl.pallas_call(
        functools.partial(_k1_body, n),
        grid=(grid,),
        in_specs=[
            pl.BlockSpec((RB, in_c), lambda i: (i, 0)),
            pl.BlockSpec((in_c, hid_c), lambda i: (0, 0)),
            pl.BlockSpec((hid_c, 1), lambda i: (0, 0)),
            pl.BlockSpec((hid_c, 1), lambda i: (0, 0)),
            pl.BlockSpec((1, in_c), lambda i: (0, 0)),
            pl.BlockSpec((1, in_c), lambda i: (0, 0)),
        ],
        out_specs=[
            pl.BlockSpec((RB, r1), lambda i: (i, 0)),
            pl.BlockSpec((RB, 1), lambda i: (i, 0)),
            pl.BlockSpec((RB, 1), lambda i: (i, 0)),
            pl.BlockSpec((RB, in_c), lambda i: (i, 0)),
            pl.BlockSpec((RB, 1), lambda i: (i, 0)),
        ],
        out_shape=[
            jax.ShapeDtypeStruct((n_pad, r1), F32),
            jax.ShapeDtypeStruct((n_pad, 1), F32),
            jax.ShapeDtypeStruct((n_pad, 1), F32),
            jax.ShapeDtypeStruct((n_pad, in_c), F32),
            jax.ShapeDtypeStruct((n_pad, 1), F32),
        ],
    )(x_p, W1, a_s1.reshape(hid_c, 1), a_d1.reshape(hid_c, 1),
      logit_p.reshape(1, in_c), embedding_w.reshape(1, in_c))


def _make_edge_kernel(n_pad, e_pad, r, with_den):
    epw = e_pad // NW
    n_chunks = epw // CH
    rps = n_pad // NS
    mesh = plsc.VectorSubcoreMesh(
        core_axis_name="c", subcore_axis_name="s",
        num_cores=NC, num_subcores=NS)

    out_type = [jax.ShapeDtypeStruct((NC, n_pad, r), F32)]
    scratch = [
        pltpu.VMEM((n_pad,), F32),
        pltpu.VMEM((n_pad,), F32),
        pltpu.VMEM((CH,), I32),
        pltpu.VMEM((CH,), I32),
        pltpu.VMEM((CH,), F32),
        pltpu.VMEM((CH, r), F32),
        pltpu.VMEM_SHARED((n_pad, r), F32),
        pltpu.SemaphoreType.DMA,
    ]
    if with_den:
        out_type.append(jax.ShapeDtypeStruct((NC, NS, n_pad), F32))
        scratch += [
            pltpu.VMEM((n_pad,), F32),
            pltpu.VMEM((32,), I32),
            pltpu.VMEM((32,), I32),
            pltpu.VMEM((16,), F32),
        ]

    def edge_body(h_hbm, es_hbm, ed_hbm, src_hbm, dst_hbm, out_hbm,
                  *rest):
        if with_den:
            (den_hbm, es_v, ed_v, src_v, dst_v, w_v, rows_v, acc, sem,
             den_v, bufp, bufn, csb) = rest
        else:
            es_v, ed_v, src_v, dst_v, w_v, rows_v, acc, sem = rest
        c = lax.axis_index("c")
        s = lax.axis_index("s")
        wid = s * NC + c
        pltpu.sync_copy(es_hbm, es_v)
        pltpu.sync_copy(ed_hbm, ed_v)
        zero16 = jnp.zeros((16,), F32)
        lanes = lax.iota(I32, 16)

        @pl.loop(0, CH)
        def _zero_rows(e):
            for j in range(r // 16):
                rows_v[e, pl.ds(j * 16, 16)] = zero16

        @pl.loop(0, rps // CH)
        def _zero_acc(b):
            pltpu.sync_copy(rows_v, acc.at[pl.ds(s * rps + b * CH, CH)])

        if with_den:
            izero16 = jnp.zeros((16,), I32)
            bufp[pl.ds(0, 16)] = izero16
            bufp[pl.ds(16, 16)] = izero16
            bufn[pl.ds(0, 16)] = izero16
            bufn[pl.ds(16, 16)] = izero16

            @pl.loop(0, n_pad // 16)
            def _zero_den(b):
                den_v[pl.ds(b * 16, 16)] = zero16

        plsc.subcore_barrier()

        @pl.loop(0, n_chunks)
        def _chunk(k):
            base = wid * epw + k * CH
            pltpu.sync_copy(src_hbm.at[pl.ds(base, CH)], src_v)
            pltpu.sync_copy(dst_hbm.at[pl.ds(base, CH)], dst_v)
            pltpu.async_copy(h_hbm.at[src_v], rows_v, sem).wait()

            @pl.loop(0, CH // 16)
            def _weights(g):
                sl = pl.ds(g * 16, 16)
                d16 = dst_v[sl]
                e1 = (plsc.load_gather(es_v, [src_v[sl]])
                      + plsc.load_gather(ed_v, [d16]))
                e2 = jnp.maximum(e1, 0.2 * e1)
                w16 = jnp.exp(e2)
                w_v[sl] = w16
                if with_den:
                    skey, sw = plsc.sort_key_val(d16, w16)
                    cs = plsc.cumsum(sw)
                    bufp[pl.ds(1, 16)] = skey
                    bufn[pl.ds(0, 16)] = skey
                    prev = bufp[pl.ds(0, 16)]
                    nxt = bufn[pl.ds(1, 16)]
                    first = (lanes == 0) | (skey != prev)
                    last = (lanes == 15) | (skey != nxt)
                    segstart = plsc.cummax(jnp.where(first, lanes, 0))
                    csb[pl.ds(0, 16)] = cs
                    basev = plsc.load_gather(
                        csb, [jnp.maximum(segstart - 1, 0)])
                    segtot = cs - jnp.where(segstart == 0, 0.0, basev)
                    plsc.addupdate_scatter(den_v, [skey], segtot, mask=last)

            @pl.loop(0, CH)
            def _scale(e):
                wsp = plsc.load_gather(w_v, [jnp.full((16,), e, I32)])
                for j in range(r // 16):
                    sl = pl.ds(j * 16, 16)
                    rows_v[e, sl] = rows_v[e, sl] * wsp

            pltpu.sync_copy(rows_v, acc.at[dst_v], add=True)

        plsc.subcore_barrier()

        @pl.loop(0, rps // CH)
        def _copy_out(b):
            off = s * rps + b * CH
            pltpu.sync_copy(acc.at[pl.ds(off, CH)],
                            out_hbm.at[c, pl.ds(off, CH)])

        if with_den:
            pltpu.sync_copy(den_v, den_hbm.at[c, s])

    return pl.kernel(
        edge_body,
        out_type=out_type if with_den else out_type[0],
        mesh=mesh,
        compiler_params=pltpu.CompilerParams(needs_layout_passes=False),
        scratch_types=scratch,
    )


def _k3_body(n, hid_c, out_c, r2, num_ref, den1_ref, b1_ref, g_ref, be_ref,
             mu_ref, var_ref, w2_ref, as2_ref, ad2_ref,
             h2e_ref, es2_ref, ed2_ref):
    i = pl.program_id(0)
    numer = num_ref[0] + num_ref[1]
    den = jnp.sum(den1_ref[...], axis=(0, 1))
    hid = numer / (den + 1e-16) + b1_ref[...]
    hid = (hid - mu_ref[...]) / jnp.sqrt(var_ref[...] + 1e-5) * g_ref[...] \
        + be_ref[...]
    hid = jnp.maximum(hid, 0.01 * hid)
    h2 = jnp.dot(hid, w2_ref[...], preferred_element_type=F32)
    es2_ref[...] = jnp.dot(h2, as2_ref[...], preferred_element_type=F32)
    ed2_ref[...] = jnp.dot(h2, ad2_ref[...], preferred_element_type=F32)
    row = i * RB + lax.broadcasted_iota(I32, (RB, 1), 0)
    ones_col = (row < n).astype(F32)
    pad = r2 - out_c - 1
    h2e_ref[...] = jnp.concatenate(
        [h2, ones_col, jnp.zeros((RB, pad), F32)], axis=1)


def _run_k3(n, n_pad, hid_c, out_c, r1, r2, num1, den1, b1, bn_gamma,
            bn_beta, bn_mean, bn_var, W2, a_s2, a_d2):
    grid = n_pad // RB
    row1 = lambda i: (i, 0)
    vec = lambda i: (0, 0)
    return pl.pallas_call(
        functools.partial(_k3_body, n, hid_c, out_c, r2),
        grid=(grid,),
        in_specs=[
            pl.BlockSpec((NC, RB, r1), lambda i: (0, i, 0)),
            pl.BlockSpec((NC, NS, RB, 1), lambda i: (0, 0, i, 0)),
            pl.BlockSpec((1, hid_c), vec),
            pl.BlockSpec((1, hid_c), vec),
            pl.BlockSpec((1, hid_c), vec),
            pl.BlockSpec((1, hid_c), vec),
            pl.BlockSpec((1, hid_c), vec),
            pl.BlockSpec((hid_c, out_c), vec),
            pl.BlockSpec((out_c, 1), vec),
            pl.BlockSpec((out_c, 1), vec),
        ],
        out_specs=[
            pl.BlockSpec((RB, r2), row1),
            pl.BlockSpec((RB, 1), row1),
            pl.BlockSpec((RB, 1), row1),
        ],
        out_shape=[
            jax.ShapeDtypeStruct((n_pad, r2), F32),
            jax.ShapeDtypeStruct((n_pad, 1), F32),
            jax.ShapeDtypeStruct((n_pad, 1), F32),
        ],
    )(num1, den1, b1.reshape(1, hid_c), bn_gamma.reshape(1, hid_c),
      bn_beta.reshape(1, hid_c), bn_mean.reshape(1, hid_c),
      bn_var.reshape(1, hid_c), W2, a_s2.reshape(out_c, 1),
      a_d2.reshape(out_c, 1))


def _k5a_body(n, out_c, num_ref, b2_ref, y_ref, tm_ref, emb_ref, s_ref,
              t_ref, loss_ref):
    i = pl.program_id(0)
    nb = num_ref[0] + num_ref[1]
    numer = nb[:, :out_c]
    den = nb[:, out_c:out_c + 1]
    o = numer / (den + 1e-16) + b2_ref[...]
    m = jnp.max(o, axis=1, keepdims=True)
    ex = jnp.exp(o - m)
    out_log = o - m - jnp.log(jnp.sum(ex, axis=1, keepdims=True))
    row = i * RB + lax.broadcasted_iota(I32, (RB, 1), 0)
    valid = (row < n).astype(F32)
    node_p = jnp.exp(out_log[:, 1:2])
    rs = valid / (s_ref[...] + 1e-6)
    pvec = node_p * rs
    nvec = (1.0 - node_p) * rs
    emb = emb_ref[...]
    tn = lax.dot_general(emb, nvec, (((0,), (0,)), ((), ())),
                         preferred_element_type=F32)
    tp = lax.dot_general(emb, pvec, (((0,), (0,)), ((), ())),
                         preferred_element_type=F32)
    tblk = jnp.concatenate([tn, tp], axis=1)
    cols = lax.broadcasted_iota(I32, (RB, out_c), 1)
    onehot = (cols == y_ref[...]).astype(F32)
    vals = jnp.sum(out_log * onehot, axis=1, keepdims=True)
    tm = tm_ref[...] * valid
    lblk = jnp.concatenate([
        jnp.sum(vals * tm).reshape(1, 1),
        jnp.sum(tm).reshape(1, 1)], axis=1)

    @pl.when(i == 0)
    def _():
        t_ref[...] = tblk
        loss_ref[...] = lblk

    @pl.when(i > 0)
    def _():
        t_ref[...] += tblk
        loss_ref[...] += lblk


def _run_k5a(n, n_pad, in_c, out_c, r2, num2, b2, y_p, tm_p, emb, s):
    grid = _rup(n, RB) // RB
    row1 = lambda i: (i, 0)
    return pl.pallas_call(
        functools.partial(_k5a_body, n, out_c),
        grid=(grid,),
        in_specs=[
            pl.BlockSpec((NC, RB, r2), lambda i: (0, i, 0)),
            pl.BlockSpec((1, out_c), lambda i: (0, 0)),
            pl.BlockSpec((RB, 1), row1),
            pl.BlockSpec((RB, 1), row1),
            pl.BlockSpec((RB, in_c), row1),
            pl.BlockSpec((RB, 1), row1),
        ],
        out_specs=[
            pl.BlockSpec((in_c, 2), lambda i: (0, 0)),
            pl.BlockSpec((1, 2), lambda i: (0, 0)),
        ],
        out_shape=[
            jax.ShapeDtypeStruct((in_c, 2), F32),
            jax.ShapeDtypeStruct((1, 2), F32),
        ],
    )(num2, b2.reshape(1, out_c), y_p, tm_p, emb, s)


def _k5b_body(emb_ref, t_ref, out_ref):
    comb = jnp.dot(emb_ref[...], t_ref[...], preferred_element_type=F32)
    m = jnp.max(comb, axis=1, keepdims=True)
    ex = jnp.exp(comb - m)
    out_ref[...] = ex / jnp.sum(ex, axis=1, keepdims=True)


def _run_k5b(n, n_pad, in_c, emb, t):
    grid = _rup(n, RB) // RB
    return pl.pallas_call(
        _k5b_body,
        grid=(grid,),
        in_specs=[
            pl.BlockSpec((RB, in_c), lambda i: (i, 0)),
            pl.BlockSpec((in_c, 2), lambda i: (0, 0)),
        ],
        out_specs=pl.BlockSpec((RB, 2), lambda i: (i, 0)),
        out_shape=jax.ShapeDtypeStruct((n, 2), F32),
    )(emb, t)


def _k5c_body(emb_i_ref, emb_j_ref, s_ref, out_ref):
    g = lax.dot_general(emb_i_ref[...], emb_j_ref[...],
                        (((1,), (1,)), ((), ())), preferred_element_type=F32)
    out_ref[...] = g * (1.0 / (s_ref[...] + 1e-6))


def _run_k5c(n, n_pad, in_c, emb, s_row):
    BM, BN = 512, 1024
    gi, gj = _rup(n, BM) // BM, _rup(n, BN) // BN
    return pl.pallas_call(
        _k5c_body,
        grid=(gi, gj),
        in_specs=[
            pl.BlockSpec((BM, in_c), lambda i, j: (i, 0)),
            pl.BlockSpec((BN, in_c), lambda i, j: (j, 0)),
            pl.BlockSpec((1, BN), lambda i, j: (0, j)),
        ],
        out_specs=pl.BlockSpec((BM, BN), lambda i, j: (i, j)),
        out_shape=jax.ShapeDtypeStruct((n, n), F32),
    )(emb, emb, s_row)


def kernel(x, edge_index, train_mask, y, W1, a_s1, a_d1, b1, bn_gamma,
           bn_beta, bn_mean, bn_var, W2, a_s2, a_d2, b2, embedding_w,
           logit_p):
    n, in_c = x.shape
    hid_c = W1.shape[1]
    out_c = W2.shape[1]
    e = edge_index.shape[1]
    e_tot = e + n
    n_pad = _rup(n, NS * CH)
    e_pad = _rup(e_tot, NW * CH)
    r1 = hid_c
    r2 = 128

    x_p = jnp.pad(x, ((0, n_pad - n), (0, 0)))
    loops = jnp.arange(n, dtype=edge_index.dtype)
    pad_idx = jnp.full((e_pad - e_tot,), n, dtype=edge_index.dtype)
    src = jnp.concatenate([edge_index[0], loops, pad_idx])
    dst = jnp.concatenate([edge_index[1], loops, pad_idx])
    y_p = jnp.pad(y, (0, n_pad - n)).reshape(n_pad, 1)
    tm_p = jnp.pad(train_mask.astype(F32), (0, n_pad - n)).reshape(n_pad, 1)

    h1e, es1, ed1, emb, s = _run_k1(
        n, n_pad, in_c, hid_c, r1, x_p, W1, a_s1, a_d1, logit_p, embedding_w)

    edge_k1 = _make_edge_kernel(n_pad, e_pad, r1, True)
    num1, den1 = edge_k1(h1e, es1.reshape(n_pad), ed1.reshape(n_pad),
                         src, dst)

    h2e, es2, ed2 = _run_k3(
        n, n_pad, hid_c, out_c, r1, r2, num1,
        den1.reshape(NC, NS, n_pad, 1), b1, bn_gamma, bn_beta,
        bn_mean, bn_var, W2, a_s2, a_d2)

    edge_k2 = _make_edge_kernel(n_pad, e_pad, r2, False)
    num2 = edge_k2(h2e, es2.reshape(n_pad), ed2.reshape(n_pad), src, dst)

    t, ls = _run_k5a(n, n_pad, in_c, out_c, r2, num2, b2, y_p, tm_p, emb, s)
    norm_node_p = _run_k5b(n, n_pad, in_c, emb, t)
    graph = _run_k5c(n, n_pad, in_c, emb, s.reshape(1, n_pad))
    loss1 = -(ls[0, 0] / ls[0, 1])
    return (norm_node_p, loss1, graph)
